# baseline (speedup 1.0000x reference)
import functools

import jax
import jax.numpy as jnp
from jax import lax
from jax.experimental import pallas as pl
from jax.experimental.pallas import tpu as pltpu
from jax.experimental.pallas import tpu_sc as plsc

NUM_ELEMENTS = 100
EMB_SIZE = 128
N_ATOMS = 100000

_NC = 2
_NS = 16
_NW = _NC * _NS
_BPW = 3128
_ILN = 3136
_CH = 128
_NCH = 25
_NBUF = 7


def _body(z_hbm, w_hbm, out_hbm, table_sh, idx_v, rows_v, g_sem, o_sem):
    wid = lax.axis_index("s") * _NC + lax.axis_index("c")
    base = wid * _BPW
    limit = jnp.minimum(base + _BPW, N_ATOMS)
    iload = jnp.minimum(base, N_ATOMS - _ILN)

    @pl.when(lax.axis_index("s") == 0)
    def _():
        pltpu.sync_copy(w_hbm, table_sh.at[pl.ds(1, NUM_ELEMENTS)])

    pltpu.sync_copy(z_hbm.at[pl.ds(iload, _ILN)], idx_v)

    plsc.subcore_barrier()

    starts = []
    for j in range(_NCH):
        starts.append(jnp.minimum(base + j * _CH, limit - _CH))

    def gather(j):
        b = j % _NBUF
        return pltpu.async_copy(
            table_sh.at[idx_v.at[pl.ds(starts[j] - iload, _CH)]],
            rows_v.at[b],
            g_sem,
        )

    def writeout(j):
        b = j % _NBUF
        return pltpu.async_copy(
            rows_v.at[b], out_hbm.at[pl.ds(starts[j], _CH)], o_sem
        )

    g_h = [None] * _NBUF
    o_h = [None] * _NBUF
    for j in range(_NCH):
        b = j % _NBUF
        if o_h[b] is not None:
            o_h[b].wait()
        g_h[b] = gather(j)
        jj = j - (_NBUF - 1)
        if jj >= 0:
            bb = jj % _NBUF
            g_h[bb].wait()
            o_h[bb] = writeout(jj)
    for jj in range(max(0, _NCH - _NBUF + 1), _NCH):
        bb = jj % _NBUF
        g_h[bb].wait()
        o_h[bb] = writeout(jj)
    for bb in range(_NBUF):
        if o_h[bb] is not None:
            o_h[bb].wait()


_embed = functools.partial(
    pl.kernel,
    out_type=jax.ShapeDtypeStruct((N_ATOMS, EMB_SIZE), jnp.float32),
    mesh=plsc.VectorSubcoreMesh(core_axis_name="c", subcore_axis_name="s"),
    scratch_types=[
        pltpu.VMEM_SHARED((NUM_ELEMENTS + 1, EMB_SIZE), jnp.float32),
        pltpu.VMEM((_ILN,), jnp.int32),
        pltpu.VMEM((_NBUF, _CH, EMB_SIZE), jnp.float32),
        pltpu.SemaphoreType.DMA,
        pltpu.SemaphoreType.DMA,
    ],
)(_body)


@jax.jit
def kernel(Z, weight):
    return _embed(Z, weight)


# speedup vs baseline: 1.0016x; 1.0016x over previous
import functools

import jax
import jax.numpy as jnp
from jax import lax
from jax.experimental import pallas as pl
from jax.experimental.pallas import tpu as pltpu
from jax.experimental.pallas import tpu_sc as plsc

NUM_ELEMENTS = 100
EMB_SIZE = 128
N_ATOMS = 100000

_NC = 2
_NS = 16
_NW = _NC * _NS
_BPW = 3128
_ILN = 3136
_CH = 128
_NCH = 25
_NBUF = 6


def _body(z_hbm, w_hbm, out_hbm, table_sh, idx_v, rows_v, g_sem, o_sem):
    wid = lax.axis_index("s") * _NC + lax.axis_index("c")
    base = wid * _BPW
    limit = jnp.minimum(base + _BPW, N_ATOMS)
    iload = jnp.minimum(base, N_ATOMS - _ILN)

    @pl.when(lax.axis_index("s") == 0)
    def _():
        pltpu.sync_copy(w_hbm, table_sh.at[pl.ds(1, NUM_ELEMENTS)])

    pltpu.sync_copy(z_hbm.at[pl.ds(iload, _ILN)], idx_v)

    plsc.subcore_barrier()

    starts = []
    for j in range(_NCH):
        starts.append(jnp.minimum(base + j * _CH, limit - _CH))

    def gather(j):
        b = j % _NBUF
        return pltpu.async_copy(
            table_sh.at[idx_v.at[pl.ds(starts[j] - iload, _CH)]],
            rows_v.at[b],
            g_sem,
        )

    def writeout(j):
        b = j % _NBUF
        return pltpu.async_copy(
            rows_v.at[b], out_hbm.at[pl.ds(starts[j], _CH)], o_sem
        )

    g_h = [None] * _NBUF
    o_h = [None] * _NBUF
    for j in range(_NCH):
        b = j % _NBUF
        if o_h[b] is not None:
            o_h[b].wait()
        g_h[b] = gather(j)
        jj = j - (_NBUF - 1)
        if jj >= 0:
            bb = jj % _NBUF
            g_h[bb].wait()
            o_h[bb] = writeout(jj)
    for jj in range(max(0, _NCH - _NBUF + 1), _NCH):
        bb = jj % _NBUF
        g_h[bb].wait()
        o_h[bb] = writeout(jj)
    for bb in range(_NBUF):
        if o_h[bb] is not None:
            o_h[bb].wait()


_embed = functools.partial(
    pl.kernel,
    out_type=jax.ShapeDtypeStruct((N_ATOMS, EMB_SIZE), jnp.float32),
    mesh=plsc.VectorSubcoreMesh(core_axis_name="c", subcore_axis_name="s"),
    scratch_types=[
        pltpu.VMEM_SHARED((NUM_ELEMENTS + 1, EMB_SIZE), jnp.float32),
        pltpu.VMEM((_ILN,), jnp.int32),
        pltpu.VMEM((_NBUF, _CH, EMB_SIZE), jnp.float32),
        pltpu.SemaphoreType.DMA,
        pltpu.SemaphoreType.DMA,
    ],
)(_body)


@jax.jit
def kernel(Z, weight):
    return _embed(Z, weight)


# speedup vs baseline: 1.0134x; 1.0118x over previous
import functools

import jax
import jax.numpy as jnp
from jax import lax
from jax.experimental import pallas as pl
from jax.experimental.pallas import tpu as pltpu
from jax.experimental.pallas import tpu_sc as plsc

NUM_ELEMENTS = 100
EMB_SIZE = 128
N_ATOMS = 100000

_NC = 2
_NS = 16
_NW = _NC * _NS
_BPW = 3128
_ILN = 3136
_CH = 128
_NCH = 25
_NBUF = 6


def _body(z_hbm, w_hbm, out_hbm, table_sh, idx_v, rows_v, g_sem, o_sem):
    wid = lax.axis_index("s") * _NC + lax.axis_index("c")
    base = wid * _BPW
    limit = jnp.minimum(base + _BPW, N_ATOMS)
    iload = jnp.minimum(base, N_ATOMS - _ILN)

    @pl.when(lax.axis_index("s") == 0)
    def _():
        pltpu.sync_copy(w_hbm, table_sh)

    pltpu.sync_copy(z_hbm.at[pl.ds(iload, _ILN)], idx_v)
    for i in range(_ILN // 16):
        sl = pl.ds(i * 16, 16)
        idx_v[sl] = idx_v[sl] - 1

    plsc.subcore_barrier()

    starts = []
    for j in range(_NCH):
        starts.append(jnp.minimum(base + j * _CH, limit - _CH))

    def gather(j):
        b = j % _NBUF
        return pltpu.async_copy(
            table_sh.at[idx_v.at[pl.ds(starts[j] - iload, _CH)]],
            rows_v.at[b],
            g_sem,
        )

    def writeout(j):
        b = j % _NBUF
        return pltpu.async_copy(
            rows_v.at[b], out_hbm.at[pl.ds(starts[j], _CH)], o_sem
        )

    g_h = [None] * _NBUF
    o_h = [None] * _NBUF
    for j in range(_NCH):
        b = j % _NBUF
        if o_h[b] is not None:
            o_h[b].wait()
        g_h[b] = gather(j)
        jj = j - (_NBUF - 1)
        if jj >= 0:
            bb = jj % _NBUF
            g_h[bb].wait()
            o_h[bb] = writeout(jj)
    for jj in range(max(0, _NCH - _NBUF + 1), _NCH):
        bb = jj % _NBUF
        g_h[bb].wait()
        o_h[bb] = writeout(jj)
    for bb in range(_NBUF):
        if o_h[bb] is not None:
            o_h[bb].wait()


_embed = functools.partial(
    pl.kernel,
    out_type=jax.ShapeDtypeStruct((N_ATOMS, EMB_SIZE), jnp.float32),
    mesh=plsc.VectorSubcoreMesh(core_axis_name="c", subcore_axis_name="s"),
    scratch_types=[
        pltpu.VMEM_SHARED((NUM_ELEMENTS, EMB_SIZE), jnp.float32),
        pltpu.VMEM((_ILN,), jnp.int32),
        pltpu.VMEM((_NBUF, _CH, EMB_SIZE), jnp.float32),
        pltpu.SemaphoreType.DMA,
        pltpu.SemaphoreType.DMA,
    ],
)(_body)


@jax.jit
def kernel(Z, weight):
    return _embed(Z, weight)
